# Initial kernel scaffold; baseline (speedup 1.0000x reference)
import jax
import jax.numpy as jnp
from jax.experimental import pallas as pl


def kernel(pred, gt):
    raise NotImplementedError("write your pallas kernel here")


# speedup vs baseline: 12.4667x; 12.4667x over previous
import functools

import jax
import jax.numpy as jnp
from jax import lax
from jax.experimental import pallas as pl
from jax.experimental.pallas import tpu as pltpu
from jax.experimental.pallas import tpu_sc as plsc

NC = 2
NS = 16
NW = NC * NS
L = 16

N_TOTAL = 16 * 512 * 512
PER_W = N_TOTAL // NW
CHUNK = 16384
N_CHUNKS = PER_W // CHUNK
STEPS = CHUNK // L

_mesh = plsc.VectorSubcoreMesh(core_axis_name="c", subcore_axis_name="s")


@functools.partial(
    pl.kernel,
    out_type=jax.ShapeDtypeStruct((NW, 3, L), jnp.int32),
    mesh=_mesh,
    scratch_types=[
        pltpu.VMEM((CHUNK,), jnp.float32),
        pltpu.VMEM((CHUNK,), jnp.int32),
        pltpu.VMEM((3, L), jnp.int32),
    ],
)
def _partials_kernel(pred_hbm, gt_hbm, out_hbm, pred_v, gt_v, part_v):
    wid = lax.axis_index("s") * NC + lax.axis_index("c")
    base = wid * PER_W

    acc1 = jnp.zeros((L,), jnp.int32)
    acc2 = jnp.zeros((L,), jnp.int32)

    for c in range(N_CHUNKS):
        off = base + c * CHUNK
        pltpu.sync_copy(pred_hbm.at[pl.ds(off, CHUNK)], pred_v)
        pltpu.sync_copy(gt_hbm.at[pl.ds(off, CHUNK)], gt_v)

        def body(i, accs):
            a1, a2 = accs
            pv = pred_v[pl.ds(i * L, L)]
            gv = gt_v[pl.ds(i * L, L)]
            p = pv > 0.0
            a1 = a1 + jnp.where(p, gv + 65536, gv)
            a2 = a2 + jnp.where(p, gv, 0)
            return a1, a2

        acc1, acc2 = lax.fori_loop(0, STEPS, body, (acc1, acc2))

    part_v[0, :] = acc1 & 0xFFFF
    part_v[1, :] = lax.shift_right_logical(acc1, 16)
    part_v[2, :] = acc2
    pltpu.sync_copy(part_v, out_hbm.at[wid])


@functools.partial(
    pl.kernel,
    out_type=jax.ShapeDtypeStruct((L,), jnp.int32),
    mesh=_mesh,
    scratch_types=[
        pltpu.VMEM((NW, 3, L), jnp.int32),
        pltpu.VMEM((L,), jnp.int32),
    ],
)
def _finalize_kernel(partials_hbm, out_hbm, buf_v, out_v):
    wid = lax.axis_index("s") * NC + lax.axis_index("c")

    @pl.when(wid == 0)
    def _():
        pltpu.sync_copy(partials_hbm, buf_v)
        sg_v = jnp.zeros((L,), jnp.int32)
        sp_v = jnp.zeros((L,), jnp.int32)
        spg_v = jnp.zeros((L,), jnp.int32)
        for r in range(NW):
            sg_v = sg_v + buf_v[r, 0, :]
            sp_v = sp_v + buf_v[r, 1, :]
            spg_v = spg_v + buf_v[r, 2, :]
        sg = sg_v[0]
        sp = sp_v[0]
        spg = spg_v[0]
        for l in range(1, L):
            sg = sg + sg_v[l]
            sp = sp + sp_v[l]
            spg = spg + spg_v[l]
        c00 = N_TOTAL - sg - sp + spg
        c01 = sp - spg
        c10 = sg - spg
        c11 = spg
        idx = lax.iota(jnp.int32, L)
        vec = jnp.where(idx == 0, c00,
              jnp.where(idx == 1, c01,
              jnp.where(idx == 2, c10,
              jnp.where(idx == 3, c11, 0))))
        out_v[...] = vec
        pltpu.sync_copy(out_v, out_hbm)


def kernel(pred, gt):
    pred_flat = pred.reshape(-1)
    gt_flat = gt.reshape(-1)
    partials = _partials_kernel(pred_flat, gt_flat)
    out16 = _finalize_kernel(partials)
    return out16[:4].reshape(2, 2)


# speedup vs baseline: 17.5819x; 1.4103x over previous
import functools

import jax
import jax.numpy as jnp
from jax import lax
from jax.experimental import pallas as pl
from jax.experimental.pallas import tpu as pltpu
from jax.experimental.pallas import tpu_sc as plsc

NC = 2
NS = 16
NW = NC * NS
L = 16

N_TOTAL = 16 * 512 * 512
PER_W = N_TOTAL // NW
CHUNK = 16384
N_CHUNKS = PER_W // CHUNK
STEPS = CHUNK // L

_mesh = plsc.VectorSubcoreMesh(core_axis_name="c", subcore_axis_name="s")


UNROLL = 8


@functools.partial(
    pl.kernel,
    out_type=jax.ShapeDtypeStruct((NW, 3, L), jnp.int32),
    mesh=_mesh,
    scratch_types=[
        pltpu.VMEM((2, CHUNK), jnp.float32),
        pltpu.VMEM((2, CHUNK), jnp.int32),
        pltpu.VMEM((3, L), jnp.int32),
        pltpu.SemaphoreType.DMA,
        pltpu.SemaphoreType.DMA,
    ],
)
def _partials_kernel(pred_hbm, gt_hbm, out_hbm, pred_v, gt_v, part_v,
                     sem0, sem1):
    wid = lax.axis_index("s") * NC + lax.axis_index("c")
    base = wid * PER_W
    sems = (sem0, sem1)

    def start(c):
        slot = c % 2
        off = base + c * CHUNK
        h1 = pltpu.async_copy(pred_hbm.at[pl.ds(off, CHUNK)], pred_v.at[slot],
                              sems[slot])
        h2 = pltpu.async_copy(gt_hbm.at[pl.ds(off, CHUNK)], gt_v.at[slot],
                              sems[slot])
        return (h1, h2)

    acc1 = jnp.zeros((L,), jnp.int32)
    acc2 = jnp.zeros((L,), jnp.int32)

    pending = start(0)
    for c in range(N_CHUNKS):
        slot = c % 2
        pending[0].wait()
        pending[1].wait()
        if c + 1 < N_CHUNKS:
            pending = start(c + 1)

        def body(i, accs):
            a1, a2 = accs
            for u in range(UNROLL):
                pv = pred_v[slot, pl.ds((i * UNROLL + u) * L, L)]
                gv = gt_v[slot, pl.ds((i * UNROLL + u) * L, L)]
                p = pv > 0.0
                a1 = a1 + jnp.where(p, gv + 65536, gv)
                a2 = a2 + jnp.where(p, gv, 0)
            return a1, a2

        acc1, acc2 = lax.fori_loop(0, STEPS // UNROLL, body, (acc1, acc2))

    part_v[0, :] = acc1 & 0xFFFF
    part_v[1, :] = lax.shift_right_logical(acc1, 16)
    part_v[2, :] = acc2
    pltpu.sync_copy(part_v, out_hbm.at[wid])


@functools.partial(
    pl.kernel,
    out_type=jax.ShapeDtypeStruct((L,), jnp.int32),
    mesh=_mesh,
    scratch_types=[
        pltpu.VMEM((NW, 3, L), jnp.int32),
        pltpu.VMEM((L,), jnp.int32),
    ],
)
def _finalize_kernel(partials_hbm, out_hbm, buf_v, out_v):
    wid = lax.axis_index("s") * NC + lax.axis_index("c")

    @pl.when(wid == 0)
    def _():
        pltpu.sync_copy(partials_hbm, buf_v)
        sg_v = jnp.zeros((L,), jnp.int32)
        sp_v = jnp.zeros((L,), jnp.int32)
        spg_v = jnp.zeros((L,), jnp.int32)
        for r in range(NW):
            sg_v = sg_v + buf_v[r, 0, :]
            sp_v = sp_v + buf_v[r, 1, :]
            spg_v = spg_v + buf_v[r, 2, :]
        sg = sg_v[0]
        sp = sp_v[0]
        spg = spg_v[0]
        for l in range(1, L):
            sg = sg + sg_v[l]
            sp = sp + sp_v[l]
            spg = spg + spg_v[l]
        c00 = N_TOTAL - sg - sp + spg
        c01 = sp - spg
        c10 = sg - spg
        c11 = spg
        idx = lax.iota(jnp.int32, L)
        vec = jnp.where(idx == 0, c00,
              jnp.where(idx == 1, c01,
              jnp.where(idx == 2, c10,
              jnp.where(idx == 3, c11, 0))))
        out_v[...] = vec
        pltpu.sync_copy(out_v, out_hbm)


def kernel(pred, gt):
    pred_flat = pred.reshape(-1)
    gt_flat = gt.reshape(-1)
    partials = _partials_kernel(pred_flat, gt_flat)
    out16 = _finalize_kernel(partials)
    return out16[:4].reshape(2, 2)


# speedup vs baseline: 30.8591x; 1.7552x over previous
import functools

import jax
import jax.numpy as jnp
from jax import lax
from jax.experimental import pallas as pl
from jax.experimental.pallas import tpu as pltpu
from jax.experimental.pallas import tpu_sc as plsc

NC = 2
NS = 16
NW = NC * NS
L = 16

B = 16
W = 512
H = 512
N_TOTAL = B * H * W
ROWS_PER_W = H // 2
ROWS_PER_CHUNK = 32
N_CHUNKS = ROWS_PER_W // ROWS_PER_CHUNK
SLICES_PER_ROW = W // L

_mesh = plsc.VectorSubcoreMesh(core_axis_name="c", subcore_axis_name="s")


@functools.partial(
    pl.kernel,
    out_type=jax.ShapeDtypeStruct((NW, 3, L), jnp.int32),
    mesh=_mesh,
    scratch_types=[
        pltpu.VMEM((2, ROWS_PER_CHUNK, W), jnp.float32),
        pltpu.VMEM((2, ROWS_PER_CHUNK, W), jnp.int32),
        pltpu.VMEM((3, L), jnp.int32),
        pltpu.SemaphoreType.DMA,
        pltpu.SemaphoreType.DMA,
    ],
)
def _partials_kernel(pred_hbm, gt_hbm, out_hbm, pred_v, gt_v, part_v,
                     sem0, sem1):
    wid = lax.axis_index("s") * NC + lax.axis_index("c")
    b = wid // 2
    r_base = (wid % 2) * ROWS_PER_W
    sems = (sem0, sem1)

    def start(c):
        slot = c % 2
        r0 = r_base + c * ROWS_PER_CHUNK
        h1 = pltpu.async_copy(
            pred_hbm.at[b, pl.ds(r0, ROWS_PER_CHUNK), :], pred_v.at[slot],
            sems[slot])
        h2 = pltpu.async_copy(
            gt_hbm.at[b, pl.ds(r0, ROWS_PER_CHUNK), :], gt_v.at[slot],
            sems[slot])
        return (h1, h2)

    acc1 = jnp.zeros((L,), jnp.int32)
    acc2 = jnp.zeros((L,), jnp.int32)

    pending = start(0)
    for c in range(N_CHUNKS):
        slot = c % 2
        pending[0].wait()
        pending[1].wait()
        if c + 1 < N_CHUNKS:
            pending = start(c + 1)

        def body(i, accs):
            a1, a2 = accs
            for u in range(SLICES_PER_ROW):
                pv = pred_v[slot, i, pl.ds(u * L, L)]
                gv = gt_v[slot, i, pl.ds(u * L, L)]
                p = pv > 0.0
                a1 = a1 + jnp.where(p, gv + 65536, gv)
                a2 = a2 + jnp.where(p, gv, 0)
            return a1, a2

        acc1, acc2 = lax.fori_loop(0, ROWS_PER_CHUNK, body, (acc1, acc2))

    part_v[0, :] = acc1 & 0xFFFF
    part_v[1, :] = lax.shift_right_logical(acc1, 16)
    part_v[2, :] = acc2
    pltpu.sync_copy(part_v, out_hbm.at[wid])


@functools.partial(
    pl.kernel,
    out_type=jax.ShapeDtypeStruct((L,), jnp.int32),
    mesh=_mesh,
    scratch_types=[
        pltpu.VMEM((NW, 3, L), jnp.int32),
        pltpu.VMEM((L,), jnp.int32),
    ],
)
def _finalize_kernel(partials_hbm, out_hbm, buf_v, out_v):
    wid = lax.axis_index("s") * NC + lax.axis_index("c")

    @pl.when(wid == 0)
    def _():
        pltpu.sync_copy(partials_hbm, buf_v)
        sg_v = jnp.zeros((L,), jnp.int32)
        sp_v = jnp.zeros((L,), jnp.int32)
        spg_v = jnp.zeros((L,), jnp.int32)
        for r in range(NW):
            sg_v = sg_v + buf_v[r, 0, :]
            sp_v = sp_v + buf_v[r, 1, :]
            spg_v = spg_v + buf_v[r, 2, :]
        sg = sg_v[0]
        sp = sp_v[0]
        spg = spg_v[0]
        for l in range(1, L):
            sg = sg + sg_v[l]
            sp = sp + sp_v[l]
            spg = spg + spg_v[l]
        c00 = N_TOTAL - sg - sp + spg
        c01 = sp - spg
        c10 = sg - spg
        c11 = spg
        idx = lax.iota(jnp.int32, L)
        vec = jnp.where(idx == 0, c00,
              jnp.where(idx == 1, c01,
              jnp.where(idx == 2, c10,
              jnp.where(idx == 3, c11, 0))))
        out_v[...] = vec
        pltpu.sync_copy(out_v, out_hbm)


def kernel(pred, gt):
    pred3 = pred.reshape(B, H, W)
    partials = _partials_kernel(pred3, gt)
    out16 = _finalize_kernel(partials)
    return out16[:4].reshape(2, 2)
